# baseline (speedup 1.0000x reference)
import functools

import jax
import jax.numpy as jnp
from jax import lax
from jax.experimental import pallas as pl
from jax.experimental.pallas import tpu as pltpu
from jax.experimental.pallas import tpu_sc as plsc

AF = 128
NF = 16
OD = 2 * AF
EPS = 1e-5


def _proj_body(atom_ref, ws_ref, wn_ref, b_ref, a_ref, p_ref):
    x = atom_ref[...]
    a_ref[...] = (
        jnp.dot(x, ws_ref[...], preferred_element_type=jnp.float32) + b_ref[...]
    )
    p_full = jnp.dot(x, wn_ref[...], preferred_element_type=jnp.float32)
    rf = p_full[:, :AF].astype(jnp.bfloat16).astype(jnp.float32)
    rc = p_full[:, AF:].astype(jnp.bfloat16).astype(jnp.float32)
    bits_f = lax.bitcast_convert_type(rf, jnp.uint32) >> 16
    bits_c = lax.bitcast_convert_type(rc, jnp.uint32) & jnp.uint32(0xFFFF0000)
    p_ref[...] = lax.bitcast_convert_type(bits_c | bits_f, jnp.int32)


def _proj(atom, w_self, w_nbr, b2d, bn):
    n = atom.shape[0]
    grid = n // bn
    return pl.pallas_call(
        _proj_body,
        grid=(grid,),
        in_specs=[
            pl.BlockSpec((bn, AF), lambda i: (i, 0)),
            pl.BlockSpec((AF, OD), lambda i: (0, 0)),
            pl.BlockSpec((AF, OD), lambda i: (0, 0)),
            pl.BlockSpec((1, OD), lambda i: (0, 0)),
        ],
        out_specs=[
            pl.BlockSpec((bn, OD), lambda i: (i, 0)),
            pl.BlockSpec((bn, AF), lambda i: (i, 0)),
        ],
        out_shape=[
            jax.ShapeDtypeStruct((n, OD), jnp.float32),
            jax.ShapeDtypeStruct((n, AF), jnp.int32),
        ],
    )(atom, w_self, w_nbr, b2d)


def _flatten_idx(idx, m):
    return idx.reshape(idx.shape[0] * m)


def _gather_rows(table, idx_flat, e0, e_chunk):
    info = plsc.get_sparse_core_info()
    nc, ns = info.num_cores, info.num_subcores
    nw = nc * ns
    per_w = e_chunk // nw
    _CH = 80 if per_w % 80 == 0 else 40
    n_ch = per_w // _CH
    mesh = plsc.VectorSubcoreMesh(core_axis_name="c", subcore_axis_name="s")

    @functools.partial(
        pl.kernel,
        mesh=mesh,
        out_type=jax.ShapeDtypeStruct((e_chunk, AF), jnp.int32),
        scratch_types=[
            pltpu.VMEM((per_w,), jnp.int32),
            pltpu.VMEM((_CH, AF), jnp.int32),
            pltpu.VMEM((_CH, AF), jnp.int32),
            pltpu.SemaphoreType.DMA,
            pltpu.SemaphoreType.DMA,
        ],
    )
    def k(p_hbm, idx_hbm, out_hbm, idx_v, rows_a, rows_b, sem_a, sem_b):
        wid = lax.axis_index("s") * nc + lax.axis_index("c")
        base = wid * per_w
        pltpu.sync_copy(idx_hbm.at[pl.ds(e0 + base, per_w)], idx_v)

        def start(t, rows, sem):
            pltpu.async_copy(p_hbm.at[idx_v.at[pl.ds(t * _CH, _CH)]], rows, sem)

        def drain(t, rows, sem):
            pltpu.make_async_copy(
                p_hbm.at[idx_v.at[pl.ds(t * _CH, _CH)]], rows, sem
            ).wait()
            pltpu.sync_copy(rows, out_hbm.at[pl.ds(base + t * _CH, _CH)])

        start(0, rows_a, sem_a)

        def body(t, carry):
            @pl.when(lax.rem(t, 2) == 0)
            def _():
                start(t + 1, rows_b, sem_b)
                drain(t, rows_a, sem_a)

            @pl.when(lax.rem(t, 2) == 1)
            def _():
                start(t + 1, rows_a, sem_a)
                drain(t, rows_b, sem_b)

            return carry

        lax.fori_loop(0, n_ch - 1, body, 0)
        if (n_ch - 1) % 2 == 0:
            drain(n_ch - 1, rows_a, sem_a)
        else:
            drain(n_ch - 1, rows_b, sem_b)

    return k(table, idx_flat)


def _unpack(g_ref):
    w = lax.bitcast_convert_type(g_ref[...], jnp.uint32)
    pf = lax.bitcast_convert_type(w << 16, jnp.float32)
    pc = lax.bitcast_convert_type(w & jnp.uint32(0xFFFF0000), jnp.float32)
    return pf, pc


def _stats_body(g_ref, nbr_ref, a_ref, we_ref, out_ref, *, bn, m):
    c = jnp.dot(nbr_ref[...], we_ref[...], preferred_element_type=jnp.float32)
    a = a_ref[...]
    arep = jnp.broadcast_to(a[:, None, :], (bn, m, OD)).reshape(bn * m, OD)
    pf, pc = _unpack(g_ref)
    gf = pf + c[:, :AF] + arep[:, :AF]
    gc = pc + c[:, AF:] + arep[:, AF:]
    out_ref[0, 0, :] = jnp.sum(gf, axis=0)
    out_ref[0, 1, :] = jnp.sum(gc, axis=0)
    out_ref[0, 2, :] = jnp.sum(gf * gf, axis=0)
    out_ref[0, 3, :] = jnp.sum(gc * gc, axis=0)


def _stats(g32, nbr_flat, a, w_edge, bn, m, nb0, n_chunk):
    grid = n_chunk // bn
    be = bn * m
    b0 = nb0 // bn
    return pl.pallas_call(
        functools.partial(_stats_body, bn=bn, m=m),
        grid=(grid,),
        in_specs=[
            pl.BlockSpec((be, AF), lambda i: (i, 0)),
            pl.BlockSpec((be, NF), lambda i, b0=b0: (i + b0, 0)),
            pl.BlockSpec((bn, OD), lambda i, b0=b0: (i + b0, 0)),
            pl.BlockSpec((NF, OD), lambda i: (0, 0)),
        ],
        out_specs=pl.BlockSpec((1, 4, AF), lambda i: (i, 0, 0)),
        out_shape=jax.ShapeDtypeStruct((grid, 4, AF), jnp.float32),
    )(g32, nbr_flat, a, w_edge)


def _apply_body(
    g_ref, nbr_ref, a_ref, we_ref, sc_ref, sh_ref, s_ref, p2_ref, *, bn, m
):
    c = jnp.dot(nbr_ref[...], we_ref[...], preferred_element_type=jnp.float32)
    a = a_ref[...]
    arep = jnp.broadcast_to(a[:, None, :], (bn, m, OD)).reshape(bn * m, OD)
    pf, pc = _unpack(g_ref)
    gf = pf + c[:, :AF] + arep[:, :AF]
    gc = pc + c[:, AF:] + arep[:, AF:]
    scv = sc_ref[...]
    shv = sh_ref[...]
    hf = gf * scv[:, :AF] + shv[:, :AF]
    hc = gc * scv[:, AF:] + shv[:, AF:]
    prod = jax.nn.sigmoid(hf) * jax.nn.softplus(hc)
    s = jnp.sum(prod.reshape(bn, m, AF), axis=1)
    s_ref[...] = s
    p2_ref[0, 0, :] = jnp.sum(s, axis=0)
    p2_ref[0, 1, :] = jnp.sum(s * s, axis=0)


def _apply(g32, nbr_flat, a, w_edge, scale2d, shift2d, bn, m, nb0, n_chunk):
    grid = n_chunk // bn
    be = bn * m
    b0 = nb0 // bn
    return pl.pallas_call(
        functools.partial(_apply_body, bn=bn, m=m),
        grid=(grid,),
        in_specs=[
            pl.BlockSpec((be, AF), lambda i: (i, 0)),
            pl.BlockSpec((be, NF), lambda i, b0=b0: (i + b0, 0)),
            pl.BlockSpec((bn, OD), lambda i, b0=b0: (i + b0, 0)),
            pl.BlockSpec((NF, OD), lambda i: (0, 0)),
            pl.BlockSpec((1, OD), lambda i: (0, 0)),
            pl.BlockSpec((1, OD), lambda i: (0, 0)),
        ],
        out_specs=[
            pl.BlockSpec((bn, AF), lambda i: (i, 0)),
            pl.BlockSpec((1, 2, AF), lambda i: (i, 0, 0)),
        ],
        out_shape=[
            jax.ShapeDtypeStruct((n_chunk, AF), jnp.float32),
            jax.ShapeDtypeStruct((grid, 2, AF), jnp.float32),
        ],
    )(g32, nbr_flat, a, w_edge, scale2d, shift2d)


def _final_body(atom_ref, s_ref, sc_ref, sh_ref, out_ref):
    out_ref[...] = jax.nn.softplus(
        atom_ref[...] + s_ref[...] * sc_ref[...] + sh_ref[...]
    )


def _final(atom, s, scale2d, shift2d, bn):
    n = atom.shape[0]
    grid = n // bn
    return pl.pallas_call(
        _final_body,
        grid=(grid,),
        in_specs=[
            pl.BlockSpec((bn, AF), lambda i: (i, 0)),
            pl.BlockSpec((bn, AF), lambda i: (i, 0)),
            pl.BlockSpec((1, AF), lambda i: (0, 0)),
            pl.BlockSpec((1, AF), lambda i: (0, 0)),
        ],
        out_specs=pl.BlockSpec((bn, AF), lambda i: (i, 0)),
        out_shape=jax.ShapeDtypeStruct((n, AF), jnp.float32),
    )(atom, s, scale2d, shift2d)


def kernel(
    atom_in_fea,
    nbr_fea,
    nbr_fea_idx,
    W_full,
    b_full,
    gamma1,
    beta1,
    gamma2,
    beta2,
):
    n, m = nbr_fea_idx.shape
    e = n * m

    w_self = W_full[:AF]
    w_nbr = W_full[AF : 2 * AF]
    w_edge = W_full[2 * AF :]
    b2d = b_full.reshape(1, OD)
    idx_flat = _flatten_idx(nbr_fea_idx, m)
    nbr_flat = nbr_fea.reshape(e, NF)

    a_mat, p_packed = _proj(atom_in_fea, w_self, w_nbr, b2d, bn=1000)

    n_chunks = 2
    n_per = n // n_chunks
    e_per = e // n_chunks
    gs = [
        _gather_rows(p_packed, idx_flat, k * e_per, e_per)
        for k in range(n_chunks)
    ]

    bn = 200
    parts = [
        _stats(gs[k], nbr_flat, a_mat, w_edge, bn, m, k * n_per, n_per)
        for k in range(n_chunks)
    ]
    partials = jnp.concatenate(parts, axis=0)
    s1 = jnp.concatenate(
        [jnp.sum(partials[:, 0, :], axis=0), jnp.sum(partials[:, 1, :], axis=0)]
    )
    s2 = jnp.concatenate(
        [jnp.sum(partials[:, 2, :], axis=0), jnp.sum(partials[:, 3, :], axis=0)]
    )
    cnt = jnp.float32(e)
    mean1 = s1 / cnt
    var1 = s2 / cnt - mean1 * mean1
    scale1 = gamma1 * lax.rsqrt(var1 + EPS)
    shift1 = beta1 - mean1 * scale1

    sc1 = scale1.reshape(1, OD)
    sh1 = shift1.reshape(1, OD)
    s_parts = []
    p2_parts = []
    for k in range(n_chunks):
        s_k, p2_k = _apply(
            gs[k], nbr_flat, a_mat, w_edge, sc1, sh1, bn, m, k * n_per, n_per
        )
        s_parts.append(s_k)
        p2_parts.append(p2_k)
    s_mat = jnp.concatenate(s_parts, axis=0)
    p2 = jnp.concatenate(p2_parts, axis=0)
    t1 = jnp.sum(p2[:, 0, :], axis=0)
    t2 = jnp.sum(p2[:, 1, :], axis=0)
    cn = jnp.float32(n)
    mean2 = t1 / cn
    var2 = t2 / cn - mean2 * mean2
    scale2 = gamma2 * lax.rsqrt(var2 + EPS)
    shift2 = beta2 - mean2 * scale2

    return _final(
        atom_in_fea, s_mat, scale2.reshape(1, AF), shift2.reshape(1, AF), bn=1000
    )


# speedup vs baseline: 1.0546x; 1.0546x over previous
import functools

import jax
import jax.numpy as jnp
from jax import lax
from jax.experimental import pallas as pl
from jax.experimental.pallas import tpu as pltpu
from jax.experimental.pallas import tpu_sc as plsc

AF = 128
NF = 16
OD = 2 * AF
EPS = 1e-5


def _proj_body(atom_ref, ws_ref, wn_ref, b_ref, a_ref, p_ref):
    x = atom_ref[...]
    a_ref[...] = (
        jnp.dot(x, ws_ref[...], preferred_element_type=jnp.float32) + b_ref[...]
    )
    p_full = jnp.dot(x, wn_ref[...], preferred_element_type=jnp.float32)
    rf = p_full[:, :AF].astype(jnp.bfloat16).astype(jnp.float32)
    rc = p_full[:, AF:].astype(jnp.bfloat16).astype(jnp.float32)
    bits_f = lax.bitcast_convert_type(rf, jnp.uint32) >> 16
    bits_c = lax.bitcast_convert_type(rc, jnp.uint32) & jnp.uint32(0xFFFF0000)
    p_ref[...] = lax.bitcast_convert_type(bits_c | bits_f, jnp.int32)


def _proj(atom, w_self, w_nbr, b2d, bn):
    n = atom.shape[0]
    grid = n // bn
    return pl.pallas_call(
        _proj_body,
        grid=(grid,),
        in_specs=[
            pl.BlockSpec((bn, AF), lambda i: (i, 0)),
            pl.BlockSpec((AF, OD), lambda i: (0, 0)),
            pl.BlockSpec((AF, OD), lambda i: (0, 0)),
            pl.BlockSpec((1, OD), lambda i: (0, 0)),
        ],
        out_specs=[
            pl.BlockSpec((bn, OD), lambda i: (i, 0)),
            pl.BlockSpec((bn, AF), lambda i: (i, 0)),
        ],
        out_shape=[
            jax.ShapeDtypeStruct((n, OD), jnp.float32),
            jax.ShapeDtypeStruct((n, AF), jnp.int32),
        ],
    )(atom, w_self, w_nbr, b2d)


def _flatten_idx(idx, m):
    return idx.reshape(idx.shape[0] * m)


def _gather_rows(table, idx_flat, e0, e_chunk):
    info = plsc.get_sparse_core_info()
    nc, ns = info.num_cores, info.num_subcores
    nw = nc * ns
    per_w = e_chunk // nw
    _CH = 80 if per_w % 80 == 0 else 40
    n_ch = per_w // _CH
    mesh = plsc.VectorSubcoreMesh(core_axis_name="c", subcore_axis_name="s")

    @functools.partial(
        pl.kernel,
        mesh=mesh,
        out_type=jax.ShapeDtypeStruct((e_chunk, AF), jnp.int32),
        scratch_types=[
            pltpu.VMEM((per_w,), jnp.int32),
            pltpu.VMEM((_CH, AF), jnp.int32),
            pltpu.VMEM((_CH, AF), jnp.int32),
            pltpu.SemaphoreType.DMA,
            pltpu.SemaphoreType.DMA,
        ],
    )
    def k(p_hbm, idx_hbm, out_hbm, idx_v, rows_a, rows_b, sem_a, sem_b):
        wid = lax.axis_index("s") * nc + lax.axis_index("c")
        base = wid * per_w
        pltpu.sync_copy(idx_hbm.at[pl.ds(e0 + base, per_w)], idx_v)

        def start(t, rows, sem):
            pltpu.async_copy(p_hbm.at[idx_v.at[pl.ds(t * _CH, _CH)]], rows, sem)

        def drain(t, rows, sem):
            pltpu.make_async_copy(
                p_hbm.at[idx_v.at[pl.ds(t * _CH, _CH)]], rows, sem
            ).wait()
            pltpu.sync_copy(rows, out_hbm.at[pl.ds(base + t * _CH, _CH)])

        start(0, rows_a, sem_a)

        def body(t, carry):
            @pl.when(lax.rem(t, 2) == 0)
            def _():
                start(t + 1, rows_b, sem_b)
                drain(t, rows_a, sem_a)

            @pl.when(lax.rem(t, 2) == 1)
            def _():
                start(t + 1, rows_a, sem_a)
                drain(t, rows_b, sem_b)

            return carry

        lax.fori_loop(0, n_ch - 1, body, 0)
        if (n_ch - 1) % 2 == 0:
            drain(n_ch - 1, rows_a, sem_a)
        else:
            drain(n_ch - 1, rows_b, sem_b)

    return k(table, idx_flat)


def _unpack(g_ref):
    w = lax.bitcast_convert_type(g_ref[...], jnp.uint32)
    pf = lax.bitcast_convert_type(w << 16, jnp.float32)
    pc = lax.bitcast_convert_type(w & jnp.uint32(0xFFFF0000), jnp.float32)
    return pf, pc


def _stats_body(g_ref, nbr_ref, a_ref, we_ref, out_ref, *, bn, m):
    c = jnp.dot(nbr_ref[...], we_ref[...], preferred_element_type=jnp.float32)
    a = a_ref[...]
    arep = jnp.broadcast_to(a[:, None, :], (bn, m, OD)).reshape(bn * m, OD)
    pf, pc = _unpack(g_ref)
    gf = pf + c[:, :AF] + arep[:, :AF]
    gc = pc + c[:, AF:] + arep[:, AF:]
    out_ref[0, 0, :] = jnp.sum(gf, axis=0)
    out_ref[0, 1, :] = jnp.sum(gc, axis=0)
    out_ref[0, 2, :] = jnp.sum(gf * gf, axis=0)
    out_ref[0, 3, :] = jnp.sum(gc * gc, axis=0)


def _stats(g32, nbr_flat, a, w_edge, bn, m, nb0, n_chunk):
    grid = n_chunk // bn
    be = bn * m
    b0 = nb0 // bn
    return pl.pallas_call(
        functools.partial(_stats_body, bn=bn, m=m),
        grid=(grid,),
        in_specs=[
            pl.BlockSpec((be, AF), lambda i: (i, 0)),
            pl.BlockSpec((be, NF), lambda i, b0=b0: (i + b0, 0)),
            pl.BlockSpec((bn, OD), lambda i, b0=b0: (i + b0, 0)),
            pl.BlockSpec((NF, OD), lambda i: (0, 0)),
        ],
        out_specs=pl.BlockSpec((1, 4, AF), lambda i: (i, 0, 0)),
        out_shape=jax.ShapeDtypeStruct((grid, 4, AF), jnp.float32),
    )(g32, nbr_flat, a, w_edge)


def _apply_body(
    g_ref, nbr_ref, a_ref, we_ref, sc_ref, sh_ref, s_ref, p2_ref, *, bn, m
):
    c = jnp.dot(nbr_ref[...], we_ref[...], preferred_element_type=jnp.float32)
    a = a_ref[...]
    arep = jnp.broadcast_to(a[:, None, :], (bn, m, OD)).reshape(bn * m, OD)
    pf, pc = _unpack(g_ref)
    gf = pf + c[:, :AF] + arep[:, :AF]
    gc = pc + c[:, AF:] + arep[:, AF:]
    scv = sc_ref[...]
    shv = sh_ref[...]
    hf = gf * scv[:, :AF] + shv[:, :AF]
    hc = gc * scv[:, AF:] + shv[:, AF:]
    prod = jax.nn.sigmoid(hf) * jax.nn.softplus(hc)
    s = jnp.sum(prod.reshape(bn, m, AF), axis=1)
    s_ref[...] = s
    p2_ref[0, 0, :] = jnp.sum(s, axis=0)
    p2_ref[0, 1, :] = jnp.sum(s * s, axis=0)


def _apply(g32, nbr_flat, a, w_edge, scale2d, shift2d, bn, m, nb0, n_chunk):
    grid = n_chunk // bn
    be = bn * m
    b0 = nb0 // bn
    return pl.pallas_call(
        functools.partial(_apply_body, bn=bn, m=m),
        grid=(grid,),
        in_specs=[
            pl.BlockSpec((be, AF), lambda i: (i, 0)),
            pl.BlockSpec((be, NF), lambda i, b0=b0: (i + b0, 0)),
            pl.BlockSpec((bn, OD), lambda i, b0=b0: (i + b0, 0)),
            pl.BlockSpec((NF, OD), lambda i: (0, 0)),
            pl.BlockSpec((1, OD), lambda i: (0, 0)),
            pl.BlockSpec((1, OD), lambda i: (0, 0)),
        ],
        out_specs=[
            pl.BlockSpec((bn, AF), lambda i: (i, 0)),
            pl.BlockSpec((1, 2, AF), lambda i: (i, 0, 0)),
        ],
        out_shape=[
            jax.ShapeDtypeStruct((n_chunk, AF), jnp.float32),
            jax.ShapeDtypeStruct((grid, 2, AF), jnp.float32),
        ],
    )(g32, nbr_flat, a, w_edge, scale2d, shift2d)


def _final_body(atom_ref, s_ref, sc_ref, sh_ref, out_ref):
    out_ref[...] = jax.nn.softplus(
        atom_ref[...] + s_ref[...] * sc_ref[...] + sh_ref[...]
    )


def _final(atom, s, scale2d, shift2d, bn):
    n = atom.shape[0]
    grid = n // bn
    return pl.pallas_call(
        _final_body,
        grid=(grid,),
        in_specs=[
            pl.BlockSpec((bn, AF), lambda i: (i, 0)),
            pl.BlockSpec((bn, AF), lambda i: (i, 0)),
            pl.BlockSpec((1, AF), lambda i: (0, 0)),
            pl.BlockSpec((1, AF), lambda i: (0, 0)),
        ],
        out_specs=pl.BlockSpec((bn, AF), lambda i: (i, 0)),
        out_shape=jax.ShapeDtypeStruct((n, AF), jnp.float32),
    )(atom, s, scale2d, shift2d)


def kernel(
    atom_in_fea,
    nbr_fea,
    nbr_fea_idx,
    W_full,
    b_full,
    gamma1,
    beta1,
    gamma2,
    beta2,
):
    n, m = nbr_fea_idx.shape
    e = n * m

    w_self = W_full[:AF]
    w_nbr = W_full[AF : 2 * AF]
    w_edge = W_full[2 * AF :]
    b2d = b_full.reshape(1, OD)
    idx_flat = _flatten_idx(nbr_fea_idx, m)
    nbr_flat = nbr_fea.reshape(e, NF)

    a_mat, p_packed = _proj(atom_in_fea, w_self, w_nbr, b2d, bn=1000)

    n_chunks = 1
    n_per = n // n_chunks
    e_per = e // n_chunks
    gs = [
        _gather_rows(p_packed, idx_flat, k * e_per, e_per)
        for k in range(n_chunks)
    ]

    bn = 200
    parts = [
        _stats(gs[k], nbr_flat, a_mat, w_edge, bn, m, k * n_per, n_per)
        for k in range(n_chunks)
    ]
    partials = jnp.concatenate(parts, axis=0)
    s1 = jnp.concatenate(
        [jnp.sum(partials[:, 0, :], axis=0), jnp.sum(partials[:, 1, :], axis=0)]
    )
    s2 = jnp.concatenate(
        [jnp.sum(partials[:, 2, :], axis=0), jnp.sum(partials[:, 3, :], axis=0)]
    )
    cnt = jnp.float32(e)
    mean1 = s1 / cnt
    var1 = s2 / cnt - mean1 * mean1
    scale1 = gamma1 * lax.rsqrt(var1 + EPS)
    shift1 = beta1 - mean1 * scale1

    sc1 = scale1.reshape(1, OD)
    sh1 = shift1.reshape(1, OD)
    s_parts = []
    p2_parts = []
    for k in range(n_chunks):
        s_k, p2_k = _apply(
            gs[k], nbr_flat, a_mat, w_edge, sc1, sh1, bn, m, k * n_per, n_per
        )
        s_parts.append(s_k)
        p2_parts.append(p2_k)
    s_mat = jnp.concatenate(s_parts, axis=0)
    p2 = jnp.concatenate(p2_parts, axis=0)
    t1 = jnp.sum(p2[:, 0, :], axis=0)
    t2 = jnp.sum(p2[:, 1, :], axis=0)
    cn = jnp.float32(n)
    mean2 = t1 / cn
    var2 = t2 / cn - mean2 * mean2
    scale2 = gamma2 * lax.rsqrt(var2 + EPS)
    shift2 = beta2 - mean2 * scale2

    return _final(
        atom_in_fea, s_mat, scale2.reshape(1, AF), shift2.reshape(1, AF), bn=1000
    )
